# baseline (speedup 1.0000x reference)
import functools

import jax
import jax.numpy as jnp
from jax import lax
from jax.experimental import pallas as pl
from jax.experimental.pallas import tpu as pltpu
from jax.experimental.pallas import tpu_sc as plsc

BATCH = 16384
EMBED = 16

_INFO = plsc.get_sparse_core_info()
NC = _INFO.num_cores
NS = _INFO.num_subcores
L = _INFO.num_lanes
NW = NC * NS
BPW = BATCH // NW
CHUNK = 128
NCHUNK = BPW // CHUNK
BLOCKS = CHUNK // L


def _sc_body(pos_u_hbm, pos_v_hbm, neg_v_hbm, w_hbm, pos_out, neg_out,
             idx_u, idx_v, idx_n, rows_u, rows_v, rows_n, acc_p, acc_n, sem):
    wid = lax.axis_index("s") * NC + lax.axis_index("c")
    base = wid * BPW
    lane = lax.iota(jnp.int32, L)

    for k in range(NCHUNK):
        cbase = base + k * CHUNK
        pltpu.sync_copy(pos_u_hbm.at[pl.ds(cbase, CHUNK)], idx_u.at[k])
        pltpu.sync_copy(pos_v_hbm.at[pl.ds(cbase, CHUNK)], idx_v.at[k])
        pltpu.sync_copy(neg_v_hbm.at[pl.ds(cbase, CHUNK)], idx_n.at[k])
        cu = pltpu.async_copy(w_hbm.at[idx_u.at[k]], rows_u, sem)
        cv = pltpu.async_copy(w_hbm.at[idx_v.at[k]], rows_v, sem)
        cn = pltpu.async_copy(w_hbm.at[idx_n.at[k]], rows_n, sem)
        cu.wait()
        cv.wait()
        cn.wait()

        lane15 = lane == (L - 1)

        def row(i, _, k=k):
            u = rows_u[i, :]
            cp = plsc.cumsum(u * rows_v[i, :])
            cn = plsc.cumsum(u * rows_n[i, :])
            pos = jnp.full((L,), k * CHUNK, jnp.int32) + i
            plsc.store_scatter(acc_p, [pos], cp, mask=lane15)
            plsc.store_scatter(acc_n, [pos], cn, mask=lane15)
            return 0

        lax.fori_loop(0, CHUNK, row, 0)

    pltpu.sync_copy(acc_p, pos_out.at[pl.ds(base, BPW)])
    pltpu.sync_copy(acc_n, neg_out.at[pl.ds(base, BPW)])


@jax.jit
def kernel(pos_u, pos_v, neg_v, W):
    pos_u = pos_u.astype(jnp.int32)
    pos_v = pos_v.astype(jnp.int32)
    neg_v = neg_v.astype(jnp.int32)
    mesh = plsc.VectorSubcoreMesh(core_axis_name="c", subcore_axis_name="s")
    f = functools.partial(
        pl.kernel,
        mesh=mesh,
        compiler_params=pltpu.CompilerParams(
            needs_layout_passes=False, use_tc_tiling_on_sc=False),
        out_type=(jax.ShapeDtypeStruct((BATCH,), jnp.float32),
                  jax.ShapeDtypeStruct((BATCH,), jnp.float32)),
        scratch_types=[
            pltpu.VMEM((NCHUNK, CHUNK), jnp.int32),
            pltpu.VMEM((NCHUNK, CHUNK), jnp.int32),
            pltpu.VMEM((NCHUNK, CHUNK), jnp.int32),
            pltpu.VMEM((CHUNK, EMBED), jnp.float32),
            pltpu.VMEM((CHUNK, EMBED), jnp.float32),
            pltpu.VMEM((CHUNK, EMBED), jnp.float32),
            pltpu.VMEM((BPW,), jnp.float32),
            pltpu.VMEM((BPW,), jnp.float32),
            pltpu.SemaphoreType.DMA,
        ],
    )(_sc_body)
    return f(pos_u, pos_v, neg_v, W)
